# baseline (speedup 1.0000x reference)
import functools

import jax
import jax.numpy as jnp
from jax import lax
from jax.experimental import pallas as pl
from jax.experimental.pallas import tpu as pltpu
from jax.experimental.pallas import tpu_sc as plsc

NU = 5000
NI = 5000
NN = 10000
D = 128
E = 2 * 160000
NC, NS, NW = 2, 16, 32
EPT = E // NW
DCH = 200
PCH = 184
NCHUNK = 56
EPTP = NCHUNK * PCH
E_PAD = NW * EPTP
NP = 10240
RPT = NP // NS
ZCH = 64
BATCH = 1024
B3 = 3 * BATCH
GW = B3 // NW

_mesh = plsc.VectorSubcoreMesh(core_axis_name="c", subcore_axis_name="s")


@functools.partial(
    pl.kernel, mesh=_mesh,
    out_type=jax.ShapeDtypeStruct((NC, NP, D), jnp.float32),
    scratch_types=[
        pltpu.VMEM((PCH,), jnp.int32),
        pltpu.VMEM((PCH,), jnp.int32),
        pltpu.VMEM((PCH,), jnp.int32),
        pltpu.VMEM((PCH,), jnp.int32),
        pltpu.VMEM((PCH, D), jnp.float32),
        pltpu.VMEM((PCH, D), jnp.float32),
        pltpu.VMEM_SHARED((NP, D), jnp.float32),
        pltpu.SemaphoreType.DMA,
        pltpu.SemaphoreType.DMA,
        pltpu.SemaphoreType.DMA,
        pltpu.SemaphoreType.DMA,
    ])
def _propagate(x_hbm, src_hbm, dst_hbm, zeros_hbm, out_hbm,
               srcA, dstA, srcB, dstB, rowsA, rowsB, acc_sh,
               semIA, semIB, semGA, semGB):
    c = lax.axis_index("c")
    s = lax.axis_index("s")
    wid = c * NS + s
    base = wid * EPTP
    K = NCHUNK // 2
    @pl.loop(0, RPT, step=ZCH)
    def _(k):
        pltpu.sync_copy(zeros_hbm.at[pl.ds(s * RPT + k, ZCH)],
                        rowsA.at[pl.ds(0, ZCH)])
        pltpu.sync_copy(rowsA.at[pl.ds(0, ZCH)],
                        acc_sh.at[pl.ds(s * RPT + k, ZCH)])
    plsc.subcore_barrier()
    pltpu.sync_copy(src_hbm.at[pl.ds(base, PCH)], srcA)
    pltpu.sync_copy(dst_hbm.at[pl.ds(base, PCH)], dstA)
    pltpu.async_copy(x_hbm.at[srcA], rowsA, semGA)
    pltpu.async_copy(src_hbm.at[pl.ds(base + PCH, PCH)], srcB, semIB)
    pltpu.async_copy(dst_hbm.at[pl.ds(base + PCH, PCH)], dstB, semIB)

    @pl.loop(0, K)
    def _(k):
        pltpu.make_async_copy(src_hbm.at[pl.ds(base, PCH)], srcB,
                              semIB).wait()
        pltpu.make_async_copy(dst_hbm.at[pl.ds(base, PCH)], dstB,
                              semIB).wait()
        pltpu.async_copy(x_hbm.at[srcB], rowsB, semGB)
        pltpu.make_async_copy(x_hbm.at[srcA], rowsA, semGA).wait()
        pltpu.sync_copy(rowsA, acc_sh.at[dstA], add=True)
        @pl.when(k < K - 1)
        def _():
            off = base + (2 * k + 2) * PCH
            pltpu.async_copy(src_hbm.at[pl.ds(off, PCH)], srcA, semIA)
            pltpu.async_copy(dst_hbm.at[pl.ds(off, PCH)], dstA, semIA)
        pltpu.make_async_copy(x_hbm.at[srcB], rowsB, semGB).wait()
        @pl.when(k < K - 1)
        def _():
            pltpu.make_async_copy(src_hbm.at[pl.ds(base, PCH)], srcA,
                                  semIA).wait()
            pltpu.make_async_copy(dst_hbm.at[pl.ds(base, PCH)], dstA,
                                  semIA).wait()
            pltpu.async_copy(x_hbm.at[srcA], rowsA, semGA)
        pltpu.sync_copy(rowsB, acc_sh.at[dstB], add=True)
        @pl.when(k < K - 1)
        def _():
            off = base + (2 * k + 3) * PCH
            pltpu.async_copy(src_hbm.at[pl.ds(off, PCH)], srcB, semIB)
            pltpu.async_copy(dst_hbm.at[pl.ds(off, PCH)], dstB, semIB)
    plsc.subcore_barrier()
    @pl.loop(0, RPT, step=ZCH)
    def _(k):
        pltpu.sync_copy(acc_sh.at[pl.ds(s * RPT + k, ZCH)],
                        rowsA.at[pl.ds(0, ZCH)])
        pltpu.sync_copy(rowsA.at[pl.ds(0, ZCH)],
                        out_hbm.at[c, pl.ds(s * RPT + k, ZCH)])


@functools.partial(
    pl.kernel, mesh=_mesh,
    out_type=jax.ShapeDtypeStruct((NC * NN,), jnp.float32),
    scratch_types=[
        pltpu.VMEM((DCH,), jnp.int32),
        pltpu.VMEM((DCH,), jnp.float32),
        pltpu.VMEM((1000,), jnp.float32),
        pltpu.VMEM_SHARED((NN,), jnp.float32),
    ])
def _degree(dst_hbm, ones_hbm, zeros1_hbm, out_hbm, dst_v, ones_v, zbuf,
            acc_sh):
    c = lax.axis_index("c")
    s = lax.axis_index("s")
    pltpu.sync_copy(ones_hbm, ones_v)
    @pl.when(s < 10)
    def _():
        pltpu.sync_copy(zeros1_hbm.at[pl.ds(s * 1000, 1000)], zbuf)
        pltpu.sync_copy(zbuf, acc_sh.at[pl.ds(s * 1000, 1000)])
    plsc.subcore_barrier()
    base = (c * NS + s) * EPT
    @pl.loop(0, EPT, step=DCH)
    def _(j):
        pltpu.sync_copy(dst_hbm.at[pl.ds(base + j, DCH)], dst_v)
        pltpu.sync_copy(ones_v, acc_sh.at[dst_v], add=True)
    plsc.subcore_barrier()
    @pl.when(s < 10)
    def _():
        pltpu.sync_copy(acc_sh.at[pl.ds(s * 1000, 1000)], zbuf)
        pltpu.sync_copy(zbuf, out_hbm.at[pl.ds(c * NN + s * 1000, 1000)])


@functools.partial(
    pl.kernel, mesh=_mesh,
    out_type=jax.ShapeDtypeStruct((B3, 4 * D), jnp.float32),
    scratch_types=[
        pltpu.VMEM((GW,), jnp.int32),
        pltpu.VMEM((GW, 4 * D), jnp.float32),
        pltpu.SemaphoreType.DMA,
    ])
def _gather_rows(emb_hbm, idx_hbm, out_hbm, idx_v, rows_v, sem):
    wid = lax.axis_index("s") * NC + lax.axis_index("c")
    base = wid * GW
    pltpu.sync_copy(idx_hbm.at[pl.ds(base, GW)], idx_v)
    pltpu.async_copy(emb_hbm.at[idx_v], rows_v, sem).wait()
    pltpu.sync_copy(rows_v, out_hbm.at[pl.ds(base, GW)])


RB = 2000


def _combine_body(s_ref, x_ref, cnt_ref, w1_ref, b1_ref, w2_ref, b2_ref,
                  o_ref):
    x = x_ref[...]
    agg = (s_ref[0] + s_ref[1] + x) / cnt_ref[...]
    t = (jnp.dot(agg, w1_ref[...], preferred_element_type=jnp.float32)
         + b1_ref[...]
         + jnp.dot(x * agg, w2_ref[...], preferred_element_type=jnp.float32)
         + b2_ref[...])
    h = jnp.where(t >= 0, t, 0.01 * t)
    nrm = jnp.maximum(jnp.sqrt(jnp.sum(h * h, axis=1, keepdims=True)), 1e-12)
    o_ref[...] = h / nrm


def _combine(S, x, cnt_col, W1, b1, W2, b2):
    return pl.pallas_call(
        _combine_body,
        grid=(NN // RB,),
        in_specs=[
            pl.BlockSpec((NC, RB, D), lambda i: (0, i, 0)),
            pl.BlockSpec((RB, D), lambda i: (i, 0)),
            pl.BlockSpec((RB, 1), lambda i: (i, 0)),
            pl.BlockSpec((D, D), lambda i: (0, 0)),
            pl.BlockSpec((1, D), lambda i: (0, 0)),
            pl.BlockSpec((D, D), lambda i: (0, 0)),
            pl.BlockSpec((1, D), lambda i: (0, 0)),
        ],
        out_specs=pl.BlockSpec((RB, D), lambda i: (i, 0)),
        out_shape=jax.ShapeDtypeStruct((NN, D), jnp.float32),
    )(S, x, cnt_col, W1, b1.reshape(1, D), W2, b2.reshape(1, D))


def _loss_body(sel_ref, o_ref):
    u = sel_ref[0:BATCH, :]
    p = sel_ref[BATCH:2 * BATCH, :]
    n = sel_ref[2 * BATCH:3 * BATCH, :]
    d = jnp.sum(u * (p - n), axis=1, keepdims=True)
    sp = jnp.maximum(-d, 0.0) + jnp.log1p(jnp.exp(-jnp.abs(d)))
    mf = jnp.sum(sp) / float(BATCH)
    reg = 1e-05 * 0.5 * (jnp.sum(u * u) + jnp.sum(p * p) + jnp.sum(n * n)) \
        / float(BATCH)
    o_ref[...] = (mf + reg).reshape(1, 1)


def _loss(sel):
    return pl.pallas_call(
        _loss_body,
        out_shape=jax.ShapeDtypeStruct((1, 1), jnp.float32),
    )(sel)


def kernel(user, pos, neg, row_u, col_i, Gu, Gi,
           W1_0, b1_0, W2_0, b2_0, W1_1, b1_1, W2_1, b2_1,
           W1_2, b1_2, W2_2, b2_2):
    row_u = row_u.astype(jnp.int32)
    col_i = col_i.astype(jnp.int32)
    cs = col_i + NU
    dst = jnp.concatenate([row_u, cs])
    src = jnp.concatenate([cs, row_u])
    zeros2 = jnp.zeros((NP, D), jnp.float32)
    zeros1 = jnp.zeros((NN,), jnp.float32)
    ones_c = jnp.ones((DCH,), jnp.float32)

    counts = _degree(dst, ones_c, zeros1).reshape(NC, NN)
    cnt_col = (counts[0] + counts[1] + 1.0)[:, None]

    x = jnp.concatenate([Gu, Gi], axis=0)
    embs = [x]
    params = [(W1_0, b1_0, W2_0, b2_0), (W1_1, b1_1, W2_1, b2_1),
              (W1_2, b1_2, W2_2, b2_2)]
    pad = E_PAD - E
    src_r = jnp.concatenate([src, jnp.zeros((pad,), jnp.int32)])
    pad_rows = NN + (jnp.arange(pad, dtype=jnp.int32) % (NP - NN))
    dst_r = jnp.concatenate([dst, pad_rows])
    for (W1, b1, W2, b2) in params:
        S = _propagate(x, src_r, dst_r, zeros2)
        x = _combine(S, x, cnt_col, W1, b1, W2, b2)
        embs.append(x)
    emb = jnp.concatenate(embs, axis=1)

    idx = jnp.concatenate([user.astype(jnp.int32),
                           NU + pos.astype(jnp.int32),
                           NU + neg.astype(jnp.int32)])
    sel = _gather_rows(emb, idx)
    return _loss(sel)[0, 0]


# speedup vs baseline: 1.0079x; 1.0079x over previous
import functools

import jax
import jax.numpy as jnp
from jax import lax
from jax.experimental import pallas as pl
from jax.experimental.pallas import tpu as pltpu
from jax.experimental.pallas import tpu_sc as plsc

NU = 5000
NI = 5000
NN = 10000
D = 128
E = 2 * 160000
NC, NS, NW = 2, 16, 32
EPT = E // NW
DCH = 200
PCH = 184
NCHUNK = 56
EPTP = NCHUNK * PCH
E_PAD = NW * EPTP
NP = 10240
RPT = NP // NS
ZCH = 64
BATCH = 1024
B3 = 3 * BATCH
GW = B3 // NW

_mesh = plsc.VectorSubcoreMesh(core_axis_name="c", subcore_axis_name="s")


@functools.partial(
    pl.kernel, mesh=_mesh,
    out_type=jax.ShapeDtypeStruct((NC, NP, D), jnp.float32),
    scratch_types=[
        pltpu.VMEM((PCH,), jnp.int32),
        pltpu.VMEM((PCH,), jnp.int32),
        pltpu.VMEM((PCH,), jnp.int32),
        pltpu.VMEM((PCH,), jnp.int32),
        pltpu.VMEM((PCH, D), jnp.float32),
        pltpu.VMEM((PCH, D), jnp.float32),
        pltpu.VMEM_SHARED((NP, D), jnp.float32),
        pltpu.SemaphoreType.DMA,
        pltpu.SemaphoreType.DMA,
        pltpu.SemaphoreType.DMA,
        pltpu.SemaphoreType.DMA,
    ])
def _propagate(x_hbm, src_hbm, dst_hbm, zeros_hbm, out_hbm,
               srcA, dstA, srcB, dstB, rowsA, rowsB, acc_sh,
               semIA, semIB, semGA, semGB):
    c = lax.axis_index("c")
    s = lax.axis_index("s")
    wid = c * NS + s
    base = wid * EPTP
    K = NCHUNK // 2
    @pl.loop(0, RPT, step=ZCH)
    def _(k):
        pltpu.sync_copy(zeros_hbm.at[pl.ds(s * RPT + k, ZCH)],
                        rowsA.at[pl.ds(0, ZCH)])
        pltpu.sync_copy(rowsA.at[pl.ds(0, ZCH)],
                        acc_sh.at[pl.ds(s * RPT + k, ZCH)])
    plsc.subcore_barrier()
    bufs = [(srcA, dstA, rowsA, semGA, semIA),
            (srcB, dstB, rowsB, semGB, semIB)]
    hg = [None] * NCHUNK
    hi = [None] * NCHUNK
    pltpu.sync_copy(src_hbm.at[pl.ds(base, PCH)], srcA)
    pltpu.sync_copy(dst_hbm.at[pl.ds(base, PCH)], dstA)
    hg[0] = pltpu.async_copy(x_hbm.at[srcA], rowsA, semGA)
    hi[1] = (pltpu.async_copy(src_hbm.at[pl.ds(base + PCH, PCH)], srcB,
                              semIB),
             pltpu.async_copy(dst_hbm.at[pl.ds(base + PCH, PCH)], dstB,
                              semIB))
    for j in range(NCHUNK):
        sbuf, dbuf, rbuf, semG, semI = bufs[j % 2]
        nsbuf, ndbuf, nrbuf, nsemG, nsemI = bufs[(j + 1) % 2]
        if j + 1 < NCHUNK:
            ha, hb = hi[j + 1]
            ha.wait()
            hb.wait()
            hg[j + 1] = pltpu.async_copy(x_hbm.at[nsbuf], nrbuf, nsemG)
        hg[j].wait()
        pltpu.sync_copy(rbuf, acc_sh.at[dbuf], add=True)
        if j + 2 < NCHUNK:
            off = base + (j + 2) * PCH
            hi[j + 2] = (pltpu.async_copy(src_hbm.at[pl.ds(off, PCH)], sbuf,
                                          semI),
                         pltpu.async_copy(dst_hbm.at[pl.ds(off, PCH)], dbuf,
                                          semI))
    plsc.subcore_barrier()
    @pl.loop(0, RPT, step=ZCH)
    def _(k):
        pltpu.sync_copy(acc_sh.at[pl.ds(s * RPT + k, ZCH)],
                        rowsA.at[pl.ds(0, ZCH)])
        pltpu.sync_copy(rowsA.at[pl.ds(0, ZCH)],
                        out_hbm.at[c, pl.ds(s * RPT + k, ZCH)])


@functools.partial(
    pl.kernel, mesh=_mesh,
    out_type=jax.ShapeDtypeStruct((NC * NN,), jnp.float32),
    scratch_types=[
        pltpu.VMEM((DCH,), jnp.int32),
        pltpu.VMEM((DCH,), jnp.float32),
        pltpu.VMEM((1000,), jnp.float32),
        pltpu.VMEM_SHARED((NN,), jnp.float32),
    ])
def _degree(dst_hbm, ones_hbm, zeros1_hbm, out_hbm, dst_v, ones_v, zbuf,
            acc_sh):
    c = lax.axis_index("c")
    s = lax.axis_index("s")
    pltpu.sync_copy(ones_hbm, ones_v)
    @pl.when(s < 10)
    def _():
        pltpu.sync_copy(zeros1_hbm.at[pl.ds(s * 1000, 1000)], zbuf)
        pltpu.sync_copy(zbuf, acc_sh.at[pl.ds(s * 1000, 1000)])
    plsc.subcore_barrier()
    base = (c * NS + s) * EPT
    @pl.loop(0, EPT, step=DCH)
    def _(j):
        pltpu.sync_copy(dst_hbm.at[pl.ds(base + j, DCH)], dst_v)
        pltpu.sync_copy(ones_v, acc_sh.at[dst_v], add=True)
    plsc.subcore_barrier()
    @pl.when(s < 10)
    def _():
        pltpu.sync_copy(acc_sh.at[pl.ds(s * 1000, 1000)], zbuf)
        pltpu.sync_copy(zbuf, out_hbm.at[pl.ds(c * NN + s * 1000, 1000)])


@functools.partial(
    pl.kernel, mesh=_mesh,
    out_type=jax.ShapeDtypeStruct((B3, 4 * D), jnp.float32),
    scratch_types=[
        pltpu.VMEM((GW,), jnp.int32),
        pltpu.VMEM((GW, 4 * D), jnp.float32),
        pltpu.SemaphoreType.DMA,
    ])
def _gather_rows(emb_hbm, idx_hbm, out_hbm, idx_v, rows_v, sem):
    wid = lax.axis_index("s") * NC + lax.axis_index("c")
    base = wid * GW
    pltpu.sync_copy(idx_hbm.at[pl.ds(base, GW)], idx_v)
    pltpu.async_copy(emb_hbm.at[idx_v], rows_v, sem).wait()
    pltpu.sync_copy(rows_v, out_hbm.at[pl.ds(base, GW)])


RB = 2000


def _combine_body(s_ref, x_ref, cnt_ref, w1_ref, b1_ref, w2_ref, b2_ref,
                  o_ref):
    x = x_ref[...]
    agg = (s_ref[0] + s_ref[1] + x) / cnt_ref[...]
    t = (jnp.dot(agg, w1_ref[...], preferred_element_type=jnp.float32)
         + b1_ref[...]
         + jnp.dot(x * agg, w2_ref[...], preferred_element_type=jnp.float32)
         + b2_ref[...])
    h = jnp.where(t >= 0, t, 0.01 * t)
    nrm = jnp.maximum(jnp.sqrt(jnp.sum(h * h, axis=1, keepdims=True)), 1e-12)
    o_ref[...] = h / nrm


def _combine(S, x, cnt_col, W1, b1, W2, b2):
    return pl.pallas_call(
        _combine_body,
        grid=(NN // RB,),
        in_specs=[
            pl.BlockSpec((NC, RB, D), lambda i: (0, i, 0)),
            pl.BlockSpec((RB, D), lambda i: (i, 0)),
            pl.BlockSpec((RB, 1), lambda i: (i, 0)),
            pl.BlockSpec((D, D), lambda i: (0, 0)),
            pl.BlockSpec((1, D), lambda i: (0, 0)),
            pl.BlockSpec((D, D), lambda i: (0, 0)),
            pl.BlockSpec((1, D), lambda i: (0, 0)),
        ],
        out_specs=pl.BlockSpec((RB, D), lambda i: (i, 0)),
        out_shape=jax.ShapeDtypeStruct((NN, D), jnp.float32),
    )(S, x, cnt_col, W1, b1.reshape(1, D), W2, b2.reshape(1, D))


def _loss_body(sel_ref, o_ref):
    u = sel_ref[0:BATCH, :]
    p = sel_ref[BATCH:2 * BATCH, :]
    n = sel_ref[2 * BATCH:3 * BATCH, :]
    d = jnp.sum(u * (p - n), axis=1, keepdims=True)
    sp = jnp.maximum(-d, 0.0) + jnp.log1p(jnp.exp(-jnp.abs(d)))
    mf = jnp.sum(sp) / float(BATCH)
    reg = 1e-05 * 0.5 * (jnp.sum(u * u) + jnp.sum(p * p) + jnp.sum(n * n)) \
        / float(BATCH)
    o_ref[...] = (mf + reg).reshape(1, 1)


def _loss(sel):
    return pl.pallas_call(
        _loss_body,
        out_shape=jax.ShapeDtypeStruct((1, 1), jnp.float32),
    )(sel)


def kernel(user, pos, neg, row_u, col_i, Gu, Gi,
           W1_0, b1_0, W2_0, b2_0, W1_1, b1_1, W2_1, b2_1,
           W1_2, b1_2, W2_2, b2_2):
    row_u = row_u.astype(jnp.int32)
    col_i = col_i.astype(jnp.int32)
    cs = col_i + NU
    dst = jnp.concatenate([row_u, cs])
    src = jnp.concatenate([cs, row_u])
    zeros2 = jnp.zeros((NP, D), jnp.float32)
    zeros1 = jnp.zeros((NN,), jnp.float32)
    ones_c = jnp.ones((DCH,), jnp.float32)

    counts = _degree(dst, ones_c, zeros1).reshape(NC, NN)
    cnt_col = (counts[0] + counts[1] + 1.0)[:, None]

    x = jnp.concatenate([Gu, Gi], axis=0)
    embs = [x]
    params = [(W1_0, b1_0, W2_0, b2_0), (W1_1, b1_1, W2_1, b2_1),
              (W1_2, b1_2, W2_2, b2_2)]
    pad = E_PAD - E
    src_r = jnp.concatenate([src, jnp.zeros((pad,), jnp.int32)])
    pad_rows = NN + (jnp.arange(pad, dtype=jnp.int32) % (NP - NN))
    dst_r = jnp.concatenate([dst, pad_rows])
    for (W1, b1, W2, b2) in params:
        S = _propagate(x, src_r, dst_r, zeros2)
        x = _combine(S, x, cnt_col, W1, b1, W2, b2)
        embs.append(x)
    emb = jnp.concatenate(embs, axis=1)

    idx = jnp.concatenate([user.astype(jnp.int32),
                           NU + pos.astype(jnp.int32),
                           NU + neg.astype(jnp.int32)])
    sel = _gather_rows(emb, idx)
    return _loss(sel)[0, 0]


# speedup vs baseline: 3.7952x; 3.7656x over previous
import functools

import jax
import jax.numpy as jnp
from jax import lax
from jax.experimental import pallas as pl
from jax.experimental.pallas import tpu as pltpu
from jax.experimental.pallas import tpu_sc as plsc

NU = 5000
NI = 5000
NN = 10000
D = 128
E = 2 * 160000
NC, NS, NW = 2, 16, 32
EPT = E // NW
DCH = 200
PCH = 184
NCHUNK = 56
EPTP = NCHUNK * PCH
E_PAD = NW * EPTP
NP = 10240
RPT = NP // NS
ZCH = 64
BATCH = 1024
B3 = 3 * BATCH
GW = B3 // NW

_mesh = plsc.VectorSubcoreMesh(core_axis_name="c", subcore_axis_name="s")


@functools.partial(
    pl.kernel, mesh=_mesh,
    out_type=jax.ShapeDtypeStruct((NC, NP, D), jnp.float32),
    scratch_types=[
        pltpu.VMEM((PCH,), jnp.int32),
        pltpu.VMEM((PCH,), jnp.int32),
        pltpu.VMEM((PCH,), jnp.int32),
        pltpu.VMEM((PCH,), jnp.int32),
        pltpu.VMEM((PCH, D), jnp.float32),
        pltpu.VMEM((PCH, D), jnp.float32),
        pltpu.VMEM_SHARED((NP, D), jnp.float32),
        pltpu.SemaphoreType.DMA,
        pltpu.SemaphoreType.DMA,
        pltpu.SemaphoreType.DMA,
        pltpu.SemaphoreType.DMA,
    ])
def _propagate(x_hbm, src_hbm, dst_hbm, zeros_hbm, out_hbm,
               srcA, dstA, srcB, dstB, rowsA, rowsB, acc_sh,
               semIA, semIB, semGA, semGB):
    c = lax.axis_index("c")
    s = lax.axis_index("s")
    wid = c * NS + s
    base = wid * EPTP
    K = NCHUNK // 2
    @pl.loop(0, RPT, step=ZCH)
    def _(k):
        pltpu.sync_copy(zeros_hbm.at[pl.ds(s * RPT + k, ZCH)],
                        rowsA.at[pl.ds(0, ZCH)])
        pltpu.sync_copy(rowsA.at[pl.ds(0, ZCH)],
                        acc_sh.at[pl.ds(s * RPT + k, ZCH)])
    plsc.subcore_barrier()
    bufs = [(srcA, dstA, rowsA, semGA, semIA),
            (srcB, dstB, rowsB, semGB, semIB)]
    hg = [None] * NCHUNK
    hi = [None] * NCHUNK
    pltpu.sync_copy(src_hbm.at[pl.ds(base, PCH)], srcA)
    pltpu.sync_copy(dst_hbm.at[pl.ds(base, PCH)], dstA)
    hg[0] = pltpu.async_copy(x_hbm.at[srcA], rowsA, semGA)
    hi[1] = (pltpu.async_copy(src_hbm.at[pl.ds(base + PCH, PCH)], srcB,
                              semIB),
             pltpu.async_copy(dst_hbm.at[pl.ds(base + PCH, PCH)], dstB,
                              semIB))
    for j in range(NCHUNK):
        sbuf, dbuf, rbuf, semG, semI = bufs[j % 2]
        nsbuf, ndbuf, nrbuf, nsemG, nsemI = bufs[(j + 1) % 2]
        if j + 1 < NCHUNK:
            ha, hb = hi[j + 1]
            ha.wait()
            hb.wait()
            hg[j + 1] = pltpu.async_copy(x_hbm.at[nsbuf], nrbuf, nsemG)
        hg[j].wait()
        pltpu.sync_copy(rbuf, acc_sh.at[dbuf], add=True)
        if j + 2 < NCHUNK:
            off = base + (j + 2) * PCH
            hi[j + 2] = (pltpu.async_copy(src_hbm.at[pl.ds(off, PCH)], sbuf,
                                          semI),
                         pltpu.async_copy(dst_hbm.at[pl.ds(off, PCH)], dbuf,
                                          semI))
    plsc.subcore_barrier()
    @pl.loop(0, RPT, step=ZCH)
    def _(k):
        pltpu.sync_copy(acc_sh.at[pl.ds(s * RPT + k, ZCH)],
                        rowsA.at[pl.ds(0, ZCH)])
        pltpu.sync_copy(rowsA.at[pl.ds(0, ZCH)],
                        out_hbm.at[c, pl.ds(s * RPT + k, ZCH)])


@functools.partial(
    pl.kernel, mesh=_mesh,
    out_type=jax.ShapeDtypeStruct((NC * NN,), jnp.float32),
    scratch_types=[
        pltpu.VMEM((DCH,), jnp.int32),
        pltpu.VMEM((DCH,), jnp.float32),
        pltpu.VMEM((1000,), jnp.float32),
        pltpu.VMEM_SHARED((NN,), jnp.float32),
    ])
def _degree(dst_hbm, ones_hbm, zeros1_hbm, out_hbm, dst_v, ones_v, zbuf,
            acc_sh):
    c = lax.axis_index("c")
    s = lax.axis_index("s")
    pltpu.sync_copy(ones_hbm, ones_v)
    @pl.when(s < 10)
    def _():
        pltpu.sync_copy(zeros1_hbm.at[pl.ds(s * 1000, 1000)], zbuf)
        pltpu.sync_copy(zbuf, acc_sh.at[pl.ds(s * 1000, 1000)])
    plsc.subcore_barrier()
    base = (c * NS + s) * EPT
    @pl.loop(0, EPT, step=DCH)
    def _(j):
        pltpu.sync_copy(dst_hbm.at[pl.ds(base + j, DCH)], dst_v)
        pltpu.sync_copy(ones_v, acc_sh.at[dst_v], add=True)
    plsc.subcore_barrier()
    @pl.when(s < 10)
    def _():
        pltpu.sync_copy(acc_sh.at[pl.ds(s * 1000, 1000)], zbuf)
        pltpu.sync_copy(zbuf, out_hbm.at[pl.ds(c * NN + s * 1000, 1000)])


@functools.partial(
    pl.kernel, mesh=_mesh,
    out_type=jax.ShapeDtypeStruct((B3, 4 * D), jnp.float32),
    scratch_types=[
        pltpu.VMEM((GW,), jnp.int32),
        pltpu.VMEM((GW, 4 * D), jnp.float32),
        pltpu.SemaphoreType.DMA,
    ])
def _gather_rows(emb_hbm, idx_hbm, out_hbm, idx_v, rows_v, sem):
    wid = lax.axis_index("s") * NC + lax.axis_index("c")
    base = wid * GW
    pltpu.sync_copy(idx_hbm.at[pl.ds(base, GW)], idx_v)
    pltpu.async_copy(emb_hbm.at[idx_v], rows_v, sem).wait()
    pltpu.sync_copy(rows_v, out_hbm.at[pl.ds(base, GW)])


RB = 2000


def _combine_body(s_ref, x_ref, cnt_ref, w1_ref, b1_ref, w2_ref, b2_ref,
                  o_ref):
    x = x_ref[...]
    agg = (s_ref[0] + s_ref[1] + x) / cnt_ref[...]
    t = (jnp.dot(agg, w1_ref[...], preferred_element_type=jnp.float32)
         + b1_ref[...]
         + jnp.dot(x * agg, w2_ref[...], preferred_element_type=jnp.float32)
         + b2_ref[...])
    h = jnp.where(t >= 0, t, 0.01 * t)
    nrm = jnp.maximum(jnp.sqrt(jnp.sum(h * h, axis=1, keepdims=True)), 1e-12)
    o_ref[...] = h / nrm


def _combine(S, x, cnt_col, W1, b1, W2, b2):
    return pl.pallas_call(
        _combine_body,
        grid=(NN // RB,),
        in_specs=[
            pl.BlockSpec((NC, RB, D), lambda i: (0, i, 0)),
            pl.BlockSpec((RB, D), lambda i: (i, 0)),
            pl.BlockSpec((RB, 1), lambda i: (i, 0)),
            pl.BlockSpec((D, D), lambda i: (0, 0)),
            pl.BlockSpec((1, D), lambda i: (0, 0)),
            pl.BlockSpec((D, D), lambda i: (0, 0)),
            pl.BlockSpec((1, D), lambda i: (0, 0)),
        ],
        out_specs=pl.BlockSpec((RB, D), lambda i: (i, 0)),
        out_shape=jax.ShapeDtypeStruct((NN, D), jnp.float32),
    )(S, x, cnt_col, W1, b1.reshape(1, D), W2, b2.reshape(1, D))


def _loss_body(sel_ref, o_ref):
    u = sel_ref[0:BATCH, :]
    p = sel_ref[BATCH:2 * BATCH, :]
    n = sel_ref[2 * BATCH:3 * BATCH, :]
    d = jnp.sum(u * (p - n), axis=1, keepdims=True)
    sp = jnp.maximum(-d, 0.0) + jnp.log1p(jnp.exp(-jnp.abs(d)))
    mf = jnp.sum(sp) / float(BATCH)
    reg = 1e-05 * 0.5 * (jnp.sum(u * u) + jnp.sum(p * p) + jnp.sum(n * n)) \
        / float(BATCH)
    o_ref[...] = (mf + reg).reshape(1, 1)


def _loss(sel):
    return pl.pallas_call(
        _loss_body,
        out_shape=jax.ShapeDtypeStruct((1, 1), jnp.float32),
    )(sel)


def kernel(user, pos, neg, row_u, col_i, Gu, Gi,
           W1_0, b1_0, W2_0, b2_0, W1_1, b1_1, W2_1, b2_1,
           W1_2, b1_2, W2_2, b2_2):
    row_u = row_u.astype(jnp.int32)
    col_i = col_i.astype(jnp.int32)
    cs = col_i + NU
    dst = jnp.concatenate([row_u, cs])
    src = jnp.concatenate([cs, row_u])
    zeros2 = jnp.zeros((NP, D), jnp.float32)
    zeros1 = jnp.zeros((NN,), jnp.float32)
    ones_c = jnp.ones((DCH,), jnp.float32)

    counts = _degree(dst, ones_c, zeros1).reshape(NC, NN)
    cnt_col = (counts[0] + counts[1] + 1.0)[:, None]

    x = jnp.concatenate([Gu, Gi], axis=0)
    embs = [x]
    params = [(W1_0, b1_0, W2_0, b2_0), (W1_1, b1_1, W2_1, b2_1),
              (W1_2, b1_2, W2_2, b2_2)]
    pad = E_PAD - E
    pad_srcs = jnp.arange(pad, dtype=jnp.int32) % NN
    src_r = jnp.concatenate([src, pad_srcs])
    pad_rows = NN + (jnp.arange(pad, dtype=jnp.int32) % (NP - NN))
    dst_r = jnp.concatenate([dst, pad_rows])
    for (W1, b1, W2, b2) in params:
        S = _propagate(x, src_r, dst_r, zeros2)
        x = _combine(S, x, cnt_col, W1, b1, W2, b2)
        embs.append(x)
    emb = jnp.concatenate(embs, axis=1)

    idx = jnp.concatenate([user.astype(jnp.int32),
                           NU + pos.astype(jnp.int32),
                           NU + neg.astype(jnp.int32)])
    sel = _gather_rows(emb, idx)
    return _loss(sel)[0, 0]


# speedup vs baseline: 3.9975x; 1.0533x over previous
import functools

import jax
import jax.numpy as jnp
from jax import lax
from jax.experimental import pallas as pl
from jax.experimental.pallas import tpu as pltpu
from jax.experimental.pallas import tpu_sc as plsc

NU = 5000
NI = 5000
NN = 10000
D = 128
E = 2 * 160000
NC, NS, NW = 2, 16, 32
EPT = E // NW
DCH = 200
PCH = 184
NCHUNK = 56
EPTP = NCHUNK * PCH
E_PAD = NW * EPTP
NP = 10240
RPT = NP // NS
ZCH = 64
BATCH = 1024
B3 = 3 * BATCH
GW = B3 // NW

_mesh = plsc.VectorSubcoreMesh(core_axis_name="c", subcore_axis_name="s")


@functools.partial(
    pl.kernel, mesh=_mesh,
    out_type=jax.ShapeDtypeStruct((NC, NP, D), jnp.float32),
    scratch_types=[
        pltpu.VMEM((PCH,), jnp.int32),
        pltpu.VMEM((PCH,), jnp.int32),
        pltpu.VMEM((PCH,), jnp.int32),
        pltpu.VMEM((PCH,), jnp.int32),
        pltpu.VMEM((PCH,), jnp.int32),
        pltpu.VMEM((PCH,), jnp.int32),
        pltpu.VMEM((PCH, D), jnp.float32),
        pltpu.VMEM((PCH, D), jnp.float32),
        pltpu.VMEM_SHARED((NP, D), jnp.float32),
        pltpu.SemaphoreType.DMA,
        pltpu.SemaphoreType.DMA,
        pltpu.SemaphoreType.DMA,
        pltpu.SemaphoreType.DMA,
        pltpu.SemaphoreType.DMA,
        pltpu.SemaphoreType.DMA,
        pltpu.SemaphoreType.DMA,
    ])
def _propagate(x_hbm, src_hbm, dst_hbm, zeros_hbm, out_hbm,
               src0, src1, src2, dst0, dst1, dst2, rowsA, rowsB, acc_sh,
               semG0, semG1, semS0, semS1, semI0, semI1, semI2):
    c = lax.axis_index("c")
    s = lax.axis_index("s")
    wid = c * NS + s
    base = wid * EPTP
    K = NCHUNK // 2
    @pl.loop(0, RPT, step=ZCH)
    def _(k):
        pltpu.sync_copy(zeros_hbm.at[pl.ds(s * RPT + k, ZCH)],
                        rowsA.at[pl.ds(0, ZCH)])
        pltpu.sync_copy(rowsA.at[pl.ds(0, ZCH)],
                        acc_sh.at[pl.ds(s * RPT + k, ZCH)])
    plsc.subcore_barrier()
    isrc = [src0, src1, src2]
    idst = [dst0, dst1, dst2]
    rows = [rowsA, rowsB]
    semG = [semG0, semG1]
    semS = [semS0, semS1]
    semI = [semI0, semI1, semI2]
    hg = [None] * NCHUNK
    hs = [None] * NCHUNK
    hi = [None] * NCHUNK
    pltpu.sync_copy(src_hbm.at[pl.ds(base, PCH)], isrc[0])
    pltpu.sync_copy(dst_hbm.at[pl.ds(base, PCH)], idst[0])
    hg[0] = pltpu.async_copy(x_hbm.at[isrc[0]], rows[0], semG[0])
    for jj in (1, 2):
        off = base + jj * PCH
        hi[jj] = (pltpu.async_copy(src_hbm.at[pl.ds(off, PCH)], isrc[jj],
                                   semI[jj]),
                  pltpu.async_copy(dst_hbm.at[pl.ds(off, PCH)], idst[jj],
                                   semI[jj]))
    for j in range(NCHUNK):
        if j + 1 < NCHUNK:
            ha, hb = hi[j + 1]
            ha.wait()
            hb.wait()
            if j >= 1:
                hs[j - 1].wait()
            hg[j + 1] = pltpu.async_copy(x_hbm.at[isrc[(j + 1) % 3]],
                                         rows[(j + 1) % 2],
                                         semG[(j + 1) % 2])
            if j + 2 < NCHUNK:
                off = base + (j + 2) * PCH
                b = (j + 2) % 3
                hi[j + 2] = (pltpu.async_copy(src_hbm.at[pl.ds(off, PCH)],
                                              isrc[b], semI[b]),
                             pltpu.async_copy(dst_hbm.at[pl.ds(off, PCH)],
                                              idst[b], semI[b]))
        hg[j].wait()
        hs[j] = pltpu.async_copy(rows[j % 2], acc_sh.at[idst[j % 3]],
                                 semS[j % 2], add=True)
    hs[NCHUNK - 2].wait()
    hs[NCHUNK - 1].wait()
    plsc.subcore_barrier()
    @pl.loop(0, RPT, step=ZCH)
    def _(k):
        pltpu.sync_copy(acc_sh.at[pl.ds(s * RPT + k, ZCH)],
                        rowsA.at[pl.ds(0, ZCH)])
        pltpu.sync_copy(rowsA.at[pl.ds(0, ZCH)],
                        out_hbm.at[c, pl.ds(s * RPT + k, ZCH)])


@functools.partial(
    pl.kernel, mesh=_mesh,
    out_type=jax.ShapeDtypeStruct((NC * NN,), jnp.float32),
    scratch_types=[
        pltpu.VMEM((DCH,), jnp.int32),
        pltpu.VMEM((DCH,), jnp.float32),
        pltpu.VMEM((1000,), jnp.float32),
        pltpu.VMEM_SHARED((NN,), jnp.float32),
    ])
def _degree(dst_hbm, ones_hbm, zeros1_hbm, out_hbm, dst_v, ones_v, zbuf,
            acc_sh):
    c = lax.axis_index("c")
    s = lax.axis_index("s")
    pltpu.sync_copy(ones_hbm, ones_v)
    @pl.when(s < 10)
    def _():
        pltpu.sync_copy(zeros1_hbm.at[pl.ds(s * 1000, 1000)], zbuf)
        pltpu.sync_copy(zbuf, acc_sh.at[pl.ds(s * 1000, 1000)])
    plsc.subcore_barrier()
    base = (c * NS + s) * EPT
    @pl.loop(0, EPT, step=DCH)
    def _(j):
        pltpu.sync_copy(dst_hbm.at[pl.ds(base + j, DCH)], dst_v)
        pltpu.sync_copy(ones_v, acc_sh.at[dst_v], add=True)
    plsc.subcore_barrier()
    @pl.when(s < 10)
    def _():
        pltpu.sync_copy(acc_sh.at[pl.ds(s * 1000, 1000)], zbuf)
        pltpu.sync_copy(zbuf, out_hbm.at[pl.ds(c * NN + s * 1000, 1000)])


@functools.partial(
    pl.kernel, mesh=_mesh,
    out_type=jax.ShapeDtypeStruct((B3, 4 * D), jnp.float32),
    scratch_types=[
        pltpu.VMEM((GW,), jnp.int32),
        pltpu.VMEM((GW, 4 * D), jnp.float32),
        pltpu.SemaphoreType.DMA,
    ])
def _gather_rows(emb_hbm, idx_hbm, out_hbm, idx_v, rows_v, sem):
    wid = lax.axis_index("s") * NC + lax.axis_index("c")
    base = wid * GW
    pltpu.sync_copy(idx_hbm.at[pl.ds(base, GW)], idx_v)
    pltpu.async_copy(emb_hbm.at[idx_v], rows_v, sem).wait()
    pltpu.sync_copy(rows_v, out_hbm.at[pl.ds(base, GW)])


RB = 2000


def _combine_body(s_ref, x_ref, cnt_ref, w1_ref, b1_ref, w2_ref, b2_ref,
                  o_ref):
    x = x_ref[...]
    agg = (s_ref[0] + s_ref[1] + x) / cnt_ref[...]
    t = (jnp.dot(agg, w1_ref[...], preferred_element_type=jnp.float32)
         + b1_ref[...]
         + jnp.dot(x * agg, w2_ref[...], preferred_element_type=jnp.float32)
         + b2_ref[...])
    h = jnp.where(t >= 0, t, 0.01 * t)
    nrm = jnp.maximum(jnp.sqrt(jnp.sum(h * h, axis=1, keepdims=True)), 1e-12)
    o_ref[...] = h / nrm


def _combine(S, x, cnt_col, W1, b1, W2, b2):
    return pl.pallas_call(
        _combine_body,
        grid=(NN // RB,),
        in_specs=[
            pl.BlockSpec((NC, RB, D), lambda i: (0, i, 0)),
            pl.BlockSpec((RB, D), lambda i: (i, 0)),
            pl.BlockSpec((RB, 1), lambda i: (i, 0)),
            pl.BlockSpec((D, D), lambda i: (0, 0)),
            pl.BlockSpec((1, D), lambda i: (0, 0)),
            pl.BlockSpec((D, D), lambda i: (0, 0)),
            pl.BlockSpec((1, D), lambda i: (0, 0)),
        ],
        out_specs=pl.BlockSpec((RB, D), lambda i: (i, 0)),
        out_shape=jax.ShapeDtypeStruct((NN, D), jnp.float32),
    )(S, x, cnt_col, W1, b1.reshape(1, D), W2, b2.reshape(1, D))


def _loss_body(sel_ref, o_ref):
    u = sel_ref[0:BATCH, :]
    p = sel_ref[BATCH:2 * BATCH, :]
    n = sel_ref[2 * BATCH:3 * BATCH, :]
    d = jnp.sum(u * (p - n), axis=1, keepdims=True)
    sp = jnp.maximum(-d, 0.0) + jnp.log1p(jnp.exp(-jnp.abs(d)))
    mf = jnp.sum(sp) / float(BATCH)
    reg = 1e-05 * 0.5 * (jnp.sum(u * u) + jnp.sum(p * p) + jnp.sum(n * n)) \
        / float(BATCH)
    o_ref[...] = (mf + reg).reshape(1, 1)


def _loss(sel):
    return pl.pallas_call(
        _loss_body,
        out_shape=jax.ShapeDtypeStruct((1, 1), jnp.float32),
    )(sel)


def kernel(user, pos, neg, row_u, col_i, Gu, Gi,
           W1_0, b1_0, W2_0, b2_0, W1_1, b1_1, W2_1, b2_1,
           W1_2, b1_2, W2_2, b2_2):
    row_u = row_u.astype(jnp.int32)
    col_i = col_i.astype(jnp.int32)
    cs = col_i + NU
    dst = jnp.concatenate([row_u, cs])
    src = jnp.concatenate([cs, row_u])
    zeros2 = jnp.zeros((NP, D), jnp.float32)
    zeros1 = jnp.zeros((NN,), jnp.float32)
    ones_c = jnp.ones((DCH,), jnp.float32)

    counts = _degree(dst, ones_c, zeros1).reshape(NC, NN)
    cnt_col = (counts[0] + counts[1] + 1.0)[:, None]

    x = jnp.concatenate([Gu, Gi], axis=0)
    embs = [x]
    params = [(W1_0, b1_0, W2_0, b2_0), (W1_1, b1_1, W2_1, b2_1),
              (W1_2, b1_2, W2_2, b2_2)]
    pad = E_PAD - E
    pad_srcs = jnp.arange(pad, dtype=jnp.int32) % NN
    src_r = jnp.concatenate([src, pad_srcs])
    pad_rows = NN + (jnp.arange(pad, dtype=jnp.int32) % (NP - NN))
    dst_r = jnp.concatenate([dst, pad_rows])
    for (W1, b1, W2, b2) in params:
        S = _propagate(x, src_r, dst_r, zeros2)
        x = _combine(S, x, cnt_col, W1, b1, W2, b2)
        embs.append(x)
    emb = jnp.concatenate(embs, axis=1)

    idx = jnp.concatenate([user.astype(jnp.int32),
                           NU + pos.astype(jnp.int32),
                           NU + neg.astype(jnp.int32)])
    sel = _gather_rows(emb, idx)
    return _loss(sel)[0, 0]
